# baseline (speedup 1.0000x reference)
import functools

import jax
import jax.numpy as jnp
from jax import lax
from jax.experimental import pallas as pl
from jax.experimental.pallas import tpu as pltpu
from jax.experimental.pallas import tpu_sc as plsc

V = 1000000
D = 64
B = 16384
NEG = 5
NC = 2
NS = 16
NW = NC * NS
BPW = B // NW
CHUNK = 128
NCHUNK = BPW // CHUNK
NGRP = CHUNK // 16
DV = D // 16
TCOL = 4096


def _transpose_body(x_ref, o_ref):
    o_ref[...] = jnp.swapaxes(x_ref[...], 0, 1)


@jax.jit
def _tc_rowmajor(tT):
    grid = (V + TCOL - 1) // TCOL
    return pl.pallas_call(
        _transpose_body,
        grid=(grid,),
        in_specs=[pl.BlockSpec((D, TCOL), lambda i: (0, i))],
        out_specs=pl.BlockSpec((TCOL, D), lambda i: (i, 0)),
        out_shape=jax.ShapeDtypeStruct((V, D), jnp.float32),
    )(tT)


def _sc_scores_kernel(pos_u_hbm, pos_v_hbm, neg_vT_hbm, u2_hbm, v2_hbm,
                      pos_out_hbm, neg_outT_hbm,
                      ku, kv, kn, hu, hv, hn,
                      rows_u0, rows_v0, rows_n0,
                      pos_sbuf, neg_sbuf, sem0):
    wid = lax.axis_index("s") * NC + lax.axis_index("c")
    base = wid * BPW
    lanes = lax.iota(jnp.int32, 16)
    bufset = (rows_u0, rows_v0, rows_n0, sem0)

    pltpu.sync_copy(pos_u_hbm.at[pl.ds(base, BPW)], hu)
    pltpu.sync_copy(pos_v_hbm.at[pl.ds(base, BPW)], hv)
    for j in range(NEG):
        pltpu.sync_copy(neg_vT_hbm.at[pl.ds(j * B + base, BPW)],
                        hn.at[pl.ds(j * BPW, BPW)])

    def split_body(g, _):
        s = pl.ds(g * 16, 16)
        r = hu[s]
        ku[s] = r >> 1
        hu[s] = (r & 1) << 6
        r = hv[s]
        kv[s] = r >> 1
        hv[s] = (r & 1) << 6
        return _

    lax.fori_loop(0, BPW // 16, split_body, 0)

    def split_n_body(g, _):
        s = pl.ds(g * 16, 16)
        r = hn[s]
        kn[s] = r >> 1
        hn[s] = (r & 1) << 6
        return _

    lax.fori_loop(0, NEG * BPW // 16, split_n_body, 0)

    def fire(c, bufset):
        ru, rv, rn, sem = bufset
        cps = [
            pltpu.async_copy(u2_hbm.at[ku.at[pl.ds(c * CHUNK, CHUNK)]],
                             ru, sem),
            pltpu.async_copy(v2_hbm.at[kv.at[pl.ds(c * CHUNK, CHUNK)]],
                             rv, sem),
        ]
        for j in range(NEG):
            cps.append(pltpu.async_copy(
                v2_hbm.at[kn.at[pl.ds(j * BPW + c * CHUNK, CHUNK)]],
                rn.at[pl.ds(j * CHUNK, CHUNK)], sem))
        return cps

    def drain(cps):
        for cp in cps:
            cp.wait()

    def compute(c, bufset):
        ru, rv, rn, _ = bufset
        start = base + c * CHUNK

        def grp_body(g, _):
            res = [jnp.zeros((16,), jnp.float32) for _ in range(1 + NEG)]
            cs = pl.ds(c * CHUNK + g * 16, 16)
            hu_g = hu[cs]
            hv_g = hv[cs]
            hn_g = [hn[pl.ds(j * BPW + c * CHUNK + g * 16, 16)]
                    for j in range(NEG)]
            for ib in range(16):
                b = g * 16 + ib
                brow = jnp.full((16,), 0, jnp.int32) + b
                lmask = lanes == ib
                zero = jnp.zeros((16,), jnp.int32)
                off_u = jnp.sum(jnp.where(lmask, hu_g, zero))
                off_v = jnp.sum(jnp.where(lmask, hv_g, zero))
                us = [plsc.load_gather(ru, [brow, off_u + k * 16 + lanes])
                      for k in range(DV)]
                vs = [plsc.load_gather(rv, [brow, off_v + k * 16 + lanes])
                      for k in range(DV)]
                pp = sum(u * v for u, v in zip(us, vs))
                res[0] = lax.select(lmask, jnp.full((16,), jnp.sum(pp)),
                                    res[0])
                for j in range(NEG):
                    off_n = jnp.sum(jnp.where(lmask, hn_g[j], zero))
                    nrow = brow + j * CHUNK
                    ns = [plsc.load_gather(
                        rn, [nrow, off_n + k * 16 + lanes])
                        for k in range(DV)]
                    nn = sum(u * nv for u, nv in zip(us, ns))
                    res[1 + j] = lax.select(
                        lmask, jnp.full((16,), jnp.sum(nn)), res[1 + j])
            pos_sbuf[pl.ds(g * 16, 16)] = res[0]
            for j in range(NEG):
                neg_sbuf[pl.ds(j * CHUNK + g * 16, 16)] = res[1 + j]
            return _

        lax.fori_loop(0, NGRP, grp_body, 0)
        pltpu.sync_copy(pos_sbuf, pos_out_hbm.at[pl.ds(start, CHUNK)])
        for j in range(NEG):
            pltpu.sync_copy(neg_sbuf.at[pl.ds(j * CHUNK, CHUNK)],
                            neg_outT_hbm.at[pl.ds(j * B + start, CHUNK)])

    def chunk_body(c, _):
        drain(fire(c, bufset))
        compute(c, bufset)
        return _

    lax.fori_loop(0, NCHUNK, chunk_body, 0)


@jax.jit
def _sc_scores(pos_u, pos_v, neg_vT, u2, v2):
    mesh = plsc.VectorSubcoreMesh(core_axis_name="c", subcore_axis_name="s")
    return pl.kernel(
        _sc_scores_kernel,
        mesh=mesh,
        compiler_params=pltpu.CompilerParams(needs_layout_passes=False),
        out_type=[
            jax.ShapeDtypeStruct((B,), jnp.float32),
            jax.ShapeDtypeStruct((NEG * B,), jnp.float32),
        ],
        scratch_types=[
            pltpu.VMEM((BPW,), jnp.int32),
            pltpu.VMEM((BPW,), jnp.int32),
            pltpu.VMEM((NEG * BPW,), jnp.int32),
            pltpu.VMEM((BPW,), jnp.int32),
            pltpu.VMEM((BPW,), jnp.int32),
            pltpu.VMEM((NEG * BPW,), jnp.int32),
            pltpu.VMEM((CHUNK, 2 * D), jnp.float32),
            pltpu.VMEM((CHUNK, 2 * D), jnp.float32),
            pltpu.VMEM((NEG * CHUNK, 2 * D), jnp.float32),
            pltpu.VMEM((CHUNK,), jnp.float32),
            pltpu.VMEM((NEG * CHUNK,), jnp.float32),
            pltpu.SemaphoreType.DMA,
        ],
    )(pos_u, pos_v, neg_vT, u2, v2)


def _loss_body(pos_ref, neg_ref, out_ref):
    p = pos_ref[...]
    n = neg_ref[...]
    ls_p = jnp.minimum(p, 0.0) - jnp.log1p(jnp.exp(-jnp.abs(p)))
    ls_n = jnp.minimum(-n, 0.0) - jnp.log1p(jnp.exp(-jnp.abs(n)))
    out_ref[0, 0] = -(jnp.sum(ls_p) + jnp.sum(ls_n))


@jax.jit
def _tc_loss(pos_s, neg_s):
    out = pl.pallas_call(
        _loss_body,
        out_shape=jax.ShapeDtypeStruct((1, 1), jnp.float32),
        out_specs=pl.BlockSpec(memory_space=pltpu.SMEM),
    )(pos_s, neg_s)
    return out[0, 0]


def kernel(pos_u, pos_v, neg_v, u_emb, v_emb):
    pos_u = pos_u.astype(jnp.int32)
    pos_v = pos_v.astype(jnp.int32)
    neg_vT = neg_v.astype(jnp.int32).T.reshape(NEG * B)
    u2 = _tc_rowmajor(u_emb.T).reshape(V // 2, 2 * D)
    v2 = _tc_rowmajor(v_emb.T).reshape(V // 2, 2 * D)
    pos_s, neg_sT = _sc_scores(pos_u, pos_v, neg_vT, u2, v2)
    return _tc_loss(pos_s.reshape(B // 128, 128),
                    neg_sT.reshape(NEG * B // 128, 128))


# speedup vs baseline: 1.0955x; 1.0955x over previous
import functools

import jax
import jax.numpy as jnp
from jax import lax
from jax.experimental import pallas as pl
from jax.experimental.pallas import tpu as pltpu
from jax.experimental.pallas import tpu_sc as plsc

V = 1000000
D = 64
B = 16384
NEG = 5
NC = 2
NS = 16
NW = NC * NS
BPW = B // NW
CHUNK = 128
NCHUNK = BPW // CHUNK
NGRP = CHUNK // 16
DV = D // 16
TCOL = 8192


def _transpose_body(x_ref, o_ref):
    eye = jnp.float32(1.0) * (lax.broadcasted_iota(jnp.int32, (D, D), 0)
                              == lax.broadcasted_iota(jnp.int32, (D, D), 1))
    o_ref[...] = lax.dot_general(
        x_ref[...], eye, (((0,), (0,)), ((), ())),
        preferred_element_type=jnp.float32)


@jax.jit
def _tc_rowmajor(tT):
    grid = (V + TCOL - 1) // TCOL
    return pl.pallas_call(
        _transpose_body,
        grid=(grid,),
        in_specs=[pl.BlockSpec((D, TCOL), lambda i: (0, i))],
        out_specs=pl.BlockSpec((TCOL, D), lambda i: (i, 0)),
        out_shape=jax.ShapeDtypeStruct((V, D), jnp.float32),
    )(tT)


def _sc_scores_kernel(pos_u_hbm, pos_v_hbm, neg_vT_hbm, u2_hbm, v2_hbm,
                      pos_out_hbm, neg_outT_hbm,
                      ku, kv, kn, hu, hv, hn,
                      rows_u0, rows_v0, rows_n0,
                      pos_sbuf, neg_sbuf, sem0):
    wid = lax.axis_index("s") * NC + lax.axis_index("c")
    base = wid * BPW
    lanes = lax.iota(jnp.int32, 16)
    bufset = (rows_u0, rows_v0, rows_n0, sem0)

    pltpu.sync_copy(pos_u_hbm.at[pl.ds(base, BPW)], hu)
    pltpu.sync_copy(pos_v_hbm.at[pl.ds(base, BPW)], hv)
    for j in range(NEG):
        pltpu.sync_copy(neg_vT_hbm.at[pl.ds(j * B + base, BPW)],
                        hn.at[pl.ds(j * BPW, BPW)])

    def split_body(g, _):
        s = pl.ds(g * 16, 16)
        r = hu[s]
        ku[s] = r >> 1
        hu[s] = (r & 1) << 6
        r = hv[s]
        kv[s] = r >> 1
        hv[s] = (r & 1) << 6
        return _

    lax.fori_loop(0, BPW // 16, split_body, 0)

    def split_n_body(g, _):
        s = pl.ds(g * 16, 16)
        r = hn[s]
        kn[s] = r >> 1
        hn[s] = (r & 1) << 6
        return _

    lax.fori_loop(0, NEG * BPW // 16, split_n_body, 0)

    def fire(c, bufset):
        ru, rv, rn, sem = bufset
        cps = [
            pltpu.async_copy(u2_hbm.at[ku.at[pl.ds(c * CHUNK, CHUNK)]],
                             ru, sem),
            pltpu.async_copy(v2_hbm.at[kv.at[pl.ds(c * CHUNK, CHUNK)]],
                             rv, sem),
        ]
        for j in range(NEG):
            cps.append(pltpu.async_copy(
                v2_hbm.at[kn.at[pl.ds(j * BPW + c * CHUNK, CHUNK)]],
                rn.at[pl.ds(j * CHUNK, CHUNK)], sem))
        return cps

    def drain(cps):
        for cp in cps:
            cp.wait()

    def compute(c, bufset):
        ru, rv, rn, _ = bufset
        start = base + c * CHUNK

        def grp_body(g, _):
            res = [jnp.zeros((16,), jnp.float32) for _ in range(1 + NEG)]
            cs = pl.ds(c * CHUNK + g * 16, 16)
            hu_g = hu[cs]
            hv_g = hv[cs]
            hn_g = [hn[pl.ds(j * BPW + c * CHUNK + g * 16, 16)]
                    for j in range(NEG)]
            for ib in range(16):
                b = g * 16 + ib
                brow = jnp.full((16,), 0, jnp.int32) + b
                lmask = lanes == ib
                zero = jnp.zeros((16,), jnp.int32)
                off_u = jnp.sum(jnp.where(lmask, hu_g, zero))
                off_v = jnp.sum(jnp.where(lmask, hv_g, zero))
                us = [plsc.load_gather(ru, [brow, off_u + k * 16 + lanes])
                      for k in range(DV)]
                vs = [plsc.load_gather(rv, [brow, off_v + k * 16 + lanes])
                      for k in range(DV)]
                pp = sum(u * v for u, v in zip(us, vs))
                res[0] = lax.select(lmask, jnp.full((16,), jnp.sum(pp)),
                                    res[0])
                for j in range(NEG):
                    off_n = jnp.sum(jnp.where(lmask, hn_g[j], zero))
                    nrow = brow + j * CHUNK
                    ns = [plsc.load_gather(
                        rn, [nrow, off_n + k * 16 + lanes])
                        for k in range(DV)]
                    nn = sum(u * nv for u, nv in zip(us, ns))
                    res[1 + j] = lax.select(
                        lmask, jnp.full((16,), jnp.sum(nn)), res[1 + j])
            pos_sbuf[pl.ds(g * 16, 16)] = res[0]
            for j in range(NEG):
                neg_sbuf[pl.ds(j * CHUNK + g * 16, 16)] = res[1 + j]
            return _

        lax.fori_loop(0, NGRP, grp_body, 0)
        pltpu.sync_copy(pos_sbuf, pos_out_hbm.at[pl.ds(start, CHUNK)])
        for j in range(NEG):
            pltpu.sync_copy(neg_sbuf.at[pl.ds(j * CHUNK, CHUNK)],
                            neg_outT_hbm.at[pl.ds(j * B + start, CHUNK)])

    def chunk_body(c, _):
        drain(fire(c, bufset))
        compute(c, bufset)
        return _

    lax.fori_loop(0, NCHUNK, chunk_body, 0)


@jax.jit
def _sc_scores(pos_u, pos_v, neg_vT, u2, v2):
    mesh = plsc.VectorSubcoreMesh(core_axis_name="c", subcore_axis_name="s")
    return pl.kernel(
        _sc_scores_kernel,
        mesh=mesh,
        compiler_params=pltpu.CompilerParams(needs_layout_passes=False),
        out_type=[
            jax.ShapeDtypeStruct((B,), jnp.float32),
            jax.ShapeDtypeStruct((NEG * B,), jnp.float32),
        ],
        scratch_types=[
            pltpu.VMEM((BPW,), jnp.int32),
            pltpu.VMEM((BPW,), jnp.int32),
            pltpu.VMEM((NEG * BPW,), jnp.int32),
            pltpu.VMEM((BPW,), jnp.int32),
            pltpu.VMEM((BPW,), jnp.int32),
            pltpu.VMEM((NEG * BPW,), jnp.int32),
            pltpu.VMEM((CHUNK, 2 * D), jnp.float32),
            pltpu.VMEM((CHUNK, 2 * D), jnp.float32),
            pltpu.VMEM((NEG * CHUNK, 2 * D), jnp.float32),
            pltpu.VMEM((CHUNK,), jnp.float32),
            pltpu.VMEM((NEG * CHUNK,), jnp.float32),
            pltpu.SemaphoreType.DMA,
        ],
    )(pos_u, pos_v, neg_vT, u2, v2)


def _loss_body(pos_ref, neg_ref, out_ref):
    p = pos_ref[...]
    n = neg_ref[...]
    ls_p = jnp.minimum(p, 0.0) - jnp.log1p(jnp.exp(-jnp.abs(p)))
    ls_n = jnp.minimum(-n, 0.0) - jnp.log1p(jnp.exp(-jnp.abs(n)))
    out_ref[0, 0] = -(jnp.sum(ls_p) + jnp.sum(ls_n))


@jax.jit
def _tc_loss(pos_s, neg_s):
    out = pl.pallas_call(
        _loss_body,
        out_shape=jax.ShapeDtypeStruct((1, 1), jnp.float32),
        out_specs=pl.BlockSpec(memory_space=pltpu.SMEM),
    )(pos_s, neg_s)
    return out[0, 0]


def kernel(pos_u, pos_v, neg_v, u_emb, v_emb):
    pos_u = pos_u.astype(jnp.int32)
    pos_v = pos_v.astype(jnp.int32)
    neg_vT = neg_v.astype(jnp.int32).T.reshape(NEG * B)
    u2 = _tc_rowmajor(u_emb.T).reshape(V // 2, 2 * D)
    v2 = _tc_rowmajor(v_emb.T).reshape(V // 2, 2 * D)
    pos_s, neg_sT = _sc_scores(pos_u, pos_v, neg_vT, u2, v2)
    return _tc_loss(pos_s.reshape(B // 128, 128),
                    neg_sT.reshape(NEG * B // 128, 128))


# speedup vs baseline: 1.1823x; 1.0792x over previous
import functools

import jax
import jax.numpy as jnp
from jax import lax
from jax.experimental import pallas as pl
from jax.experimental.pallas import tpu as pltpu
from jax.experimental.pallas import tpu_sc as plsc

V = 1000000
D = 64
B = 16384
NEG = 5
NC = 2
NS = 16
NW = NC * NS
BPW = B // NW
CHUNK = 128
NCHUNK = BPW // CHUNK
NGRP = CHUNK // 16
DV = D // 16
TCOL = 8192


def _transpose_body(x_ref, o_ref):
    eye = jnp.float32(1.0) * (lax.broadcasted_iota(jnp.int32, (D, D), 0)
                              == lax.broadcasted_iota(jnp.int32, (D, D), 1))
    o_ref[...] = lax.dot_general(
        x_ref[...], eye, (((0,), (0,)), ((), ())),
        preferred_element_type=jnp.float32)


@jax.jit
def _tc_rowmajor(tT):
    grid = (V + TCOL - 1) // TCOL
    return pl.pallas_call(
        _transpose_body,
        grid=(grid,),
        in_specs=[pl.BlockSpec((D, TCOL), lambda i: (0, i))],
        out_specs=pl.BlockSpec((TCOL, D), lambda i: (i, 0)),
        out_shape=jax.ShapeDtypeStruct((V, D), jnp.float32),
    )(tT)


def _sc_scores_kernel(pos_u_hbm, pos_v_hbm, neg_vT_hbm, u2_hbm, v2_hbm,
                      pos_out_hbm, neg_outT_hbm,
                      ku, kv, kn, hu, hv, hn,
                      rows_u0, rows_v0, rows_n0,
                      pos_sbuf, neg_sbuf, sem0):
    wid = lax.axis_index("s") * NC + lax.axis_index("c")
    base = wid * BPW
    lanes = lax.iota(jnp.int32, 16)
    bufset = (rows_u0, rows_v0, rows_n0, sem0)

    pltpu.sync_copy(pos_u_hbm.at[pl.ds(base, BPW)], hu)
    pltpu.sync_copy(pos_v_hbm.at[pl.ds(base, BPW)], hv)
    for j in range(NEG):
        pltpu.sync_copy(neg_vT_hbm.at[pl.ds(j * B + base, BPW)],
                        hn.at[pl.ds(j * BPW, BPW)])

    def split_body(g, _):
        s = pl.ds(g * 16, 16)
        r = hu[s]
        ku[s] = r >> 1
        hu[s] = (r & 1) << 6
        r = hv[s]
        kv[s] = r >> 1
        hv[s] = (r & 1) << 6
        return _

    lax.fori_loop(0, BPW // 16, split_body, 0)

    def split_n_body(g, _):
        s = pl.ds(g * 16, 16)
        r = hn[s]
        kn[s] = r >> 1
        hn[s] = (r & 1) << 6
        return _

    lax.fori_loop(0, NEG * BPW // 16, split_n_body, 0)

    def fire(c, bufset):
        ru, rv, rn, sem = bufset
        cps = [
            pltpu.async_copy(u2_hbm.at[ku.at[pl.ds(c * CHUNK, CHUNK)]],
                             ru, sem),
            pltpu.async_copy(v2_hbm.at[kv.at[pl.ds(c * CHUNK, CHUNK)]],
                             rv, sem),
        ]
        for j in range(NEG):
            cps.append(pltpu.async_copy(
                v2_hbm.at[kn.at[pl.ds(j * BPW + c * CHUNK, CHUNK)]],
                rn.at[pl.ds(j * CHUNK, CHUNK)], sem))
        return cps

    def drain(cps):
        for cp in cps:
            cp.wait()

    def compute(c, bufset):
        ru, rv, rn, _ = bufset
        start = base + c * CHUNK

        def grp_body(g, _):
            res = [jnp.zeros((16,), jnp.float32) for _ in range(1 + NEG)]
            cs = pl.ds(c * CHUNK + g * 16, 16)
            hu_g = hu[cs]
            hv_g = hv[cs]
            hn_g = [hn[pl.ds(j * BPW + c * CHUNK + g * 16, 16)]
                    for j in range(NEG)]
            for ib in range(16):
                b = g * 16 + ib
                brow = jnp.full((16,), 0, jnp.int32) + b
                lmask = lanes == ib
                zero = jnp.zeros((16,), jnp.int32)
                off_u = jnp.sum(jnp.where(lmask, hu_g, zero))
                off_v = jnp.sum(jnp.where(lmask, hv_g, zero))
                us = [plsc.load_gather(ru, [brow, off_u + k * 16 + lanes])
                      for k in range(DV)]
                vs = [plsc.load_gather(rv, [brow, off_v + k * 16 + lanes])
                      for k in range(DV)]
                pp = sum(u * v for u, v in zip(us, vs))
                res[0] = lax.select(lmask, jnp.full((16,), jnp.sum(pp)),
                                    res[0])
                for j in range(NEG):
                    off_n = jnp.sum(jnp.where(lmask, hn_g[j], zero))
                    nrow = brow + j * CHUNK
                    ns = [plsc.load_gather(
                        rn, [nrow, off_n + k * 16 + lanes])
                        for k in range(DV)]
                    nn = sum(u * nv for u, nv in zip(us, ns))
                    res[1 + j] = lax.select(
                        lmask, jnp.full((16,), jnp.sum(nn)), res[1 + j])
            pos_sbuf[pl.ds(g * 16, 16)] = res[0]
            for j in range(NEG):
                neg_sbuf[pl.ds(j * CHUNK + g * 16, 16)] = res[1 + j]
            return _

        lax.fori_loop(0, NGRP, grp_body, 0)
        pltpu.sync_copy(pos_sbuf, pos_out_hbm.at[pl.ds(start, CHUNK)])
        for j in range(NEG):
            pltpu.sync_copy(neg_sbuf.at[pl.ds(j * CHUNK, CHUNK)],
                            neg_outT_hbm.at[pl.ds(j * B + start, CHUNK)])

    def chunk_body(c, _):
        drain(fire(c, bufset))
        compute(c, bufset)
        return _

    lax.fori_loop(0, NCHUNK, chunk_body, 0)


@jax.jit
def _sc_scores(pos_u, pos_v, neg_vT, u2, v2):
    mesh = plsc.VectorSubcoreMesh(core_axis_name="c", subcore_axis_name="s")
    return pl.kernel(
        _sc_scores_kernel,
        mesh=mesh,
        compiler_params=pltpu.CompilerParams(needs_layout_passes=False),
        out_type=[
            jax.ShapeDtypeStruct((B,), jnp.float32),
            jax.ShapeDtypeStruct((NEG * B,), jnp.float32),
        ],
        scratch_types=[
            pltpu.VMEM((BPW,), jnp.int32),
            pltpu.VMEM((BPW,), jnp.int32),
            pltpu.VMEM((NEG * BPW,), jnp.int32),
            pltpu.VMEM((BPW,), jnp.int32),
            pltpu.VMEM((BPW,), jnp.int32),
            pltpu.VMEM((NEG * BPW,), jnp.int32),
            pltpu.VMEM((CHUNK, 2 * D), jnp.float32),
            pltpu.VMEM((CHUNK, 2 * D), jnp.float32),
            pltpu.VMEM((NEG * CHUNK, 2 * D), jnp.float32),
            pltpu.VMEM((CHUNK,), jnp.float32),
            pltpu.VMEM((NEG * CHUNK,), jnp.float32),
            pltpu.SemaphoreType.DMA,
        ],
    )(pos_u, pos_v, neg_vT, u2, v2)


def _loss_body(pos_ref, neg_ref, out_ref):
    p = pos_ref[...]
    n = neg_ref[...]
    ls_p = jnp.minimum(p, 0.0) - jnp.log1p(jnp.exp(-jnp.abs(p)))
    ls_n = jnp.minimum(-n, 0.0) - jnp.log1p(jnp.exp(-jnp.abs(n)))
    out_ref[0, 0] = -(jnp.sum(ls_p) + jnp.sum(ls_n))


@jax.jit
def _tc_loss(pos_s, neg_s):
    out = pl.pallas_call(
        _loss_body,
        out_shape=jax.ShapeDtypeStruct((1, 1), jnp.float32),
        out_specs=pl.BlockSpec(memory_space=pltpu.SMEM),
    )(pos_s, neg_s)
    return out[0, 0]


def kernel(pos_u, pos_v, neg_v, u_emb, v_emb):
    pos_u = pos_u.astype(jnp.int32)
    pos_v = pos_v.astype(jnp.int32)
    neg_vT = neg_v.astype(jnp.int32).T.reshape(NEG * B)
    u2 = _tc_rowmajor(u_emb.T).reshape(V // 2, 2 * D)
    v2 = v_emb.reshape(V // 2, 2 * D)
    pos_s, neg_sT = _sc_scores(pos_u, pos_v, neg_vT, u2, v2)
    return _tc_loss(pos_s.reshape(B // 128, 128),
                    neg_sT.reshape(NEG * B // 128, 128))


# speedup vs baseline: 1.3397x; 1.1332x over previous
import functools

import jax
import jax.numpy as jnp
from jax import lax
from jax.experimental import pallas as pl
from jax.experimental.pallas import tpu as pltpu
from jax.experimental.pallas import tpu_sc as plsc

V = 1000000
D = 64
B = 16384
NEG = 5
NC = 2
NS = 16
NW = NC * NS
BPW = B // NW
CHUNK = 128
NCHUNK = BPW // CHUNK
NGRP = CHUNK // 16
DV = D // 16


def _sc_scores_kernel(pos_u_hbm, pos_v_hbm, neg_vT_hbm, u_emb_hbm, v_emb_hbm,
                      pos_out_hbm, neg_outT_hbm,
                      idx_u, idx_v, idx_n,
                      rows_u0, rows_v0, rows_n0,
                      rows_u1, rows_v1, rows_n1,
                      pos_sbuf, neg_sbuf, sem0, sem1):
    wid = lax.axis_index("s") * NC + lax.axis_index("c")
    base = wid * BPW
    lanes = lax.iota(jnp.int32, 16)
    bufs = ((rows_u0, rows_v0, rows_n0, sem0),
            (rows_u1, rows_v1, rows_n1, sem1))

    pltpu.sync_copy(pos_u_hbm.at[pl.ds(base, BPW)], idx_u)
    pltpu.sync_copy(pos_v_hbm.at[pl.ds(base, BPW)], idx_v)
    for j in range(NEG):
        pltpu.sync_copy(neg_vT_hbm.at[pl.ds(j * B + base, BPW)],
                        idx_n.at[pl.ds(j * BPW, BPW)])

    def fire(c, bufset):
        ru, rv, rn, sem = bufset
        cps = [
            pltpu.async_copy(u_emb_hbm.at[idx_u.at[pl.ds(c * CHUNK, CHUNK)]],
                             ru, sem),
            pltpu.async_copy(v_emb_hbm.at[idx_v.at[pl.ds(c * CHUNK, CHUNK)]],
                             rv, sem),
        ]
        for j in range(NEG):
            cps.append(pltpu.async_copy(
                v_emb_hbm.at[idx_n.at[pl.ds(j * BPW + c * CHUNK, CHUNK)]],
                rn.at[pl.ds(j * CHUNK, CHUNK)], sem))
        return cps

    def compute(c, bufset):
        ru, rv, rn, _ = bufset
        start = base + c * CHUNK

        def grp_body(g, _):
            res = [jnp.zeros((16,), jnp.float32) for _ in range(1 + NEG)]
            for ib in range(16):
                b = g * 16 + ib
                lmask = lanes == ib
                us = [ru[b, pl.ds(k * 16, 16)] for k in range(DV)]
                vs = [rv[b, pl.ds(k * 16, 16)] for k in range(DV)]
                pp = sum(u * v for u, v in zip(us, vs))
                res[0] = lax.select(lmask, jnp.full((16,), jnp.sum(pp)),
                                    res[0])
                for j in range(NEG):
                    ns = [rn[j * CHUNK + b, pl.ds(k * 16, 16)]
                          for k in range(DV)]
                    nn = sum(u * nv for u, nv in zip(us, ns))
                    res[1 + j] = lax.select(
                        lmask, jnp.full((16,), jnp.sum(nn)), res[1 + j])
            pos_sbuf[pl.ds(g * 16, 16)] = res[0]
            for j in range(NEG):
                neg_sbuf[pl.ds(j * CHUNK + g * 16, 16)] = res[1 + j]
            return _

        lax.fori_loop(0, NGRP, grp_body, 0)
        pltpu.sync_copy(pos_sbuf, pos_out_hbm.at[pl.ds(start, CHUNK)])
        for j in range(NEG):
            pltpu.sync_copy(neg_sbuf.at[pl.ds(j * CHUNK, CHUNK)],
                            neg_outT_hbm.at[pl.ds(j * B + start, CHUNK)])

    pending = fire(0, bufs[0])
    for c in range(NCHUNK):
        nxt = fire(c + 1, bufs[(c + 1) % 2]) if c + 1 < NCHUNK else None
        for cp in pending:
            cp.wait()
        compute(c, bufs[c % 2])
        pending = nxt


@jax.jit
def _sc_scores(pos_u, pos_v, neg_vT, u_emb, v_emb):
    mesh = plsc.VectorSubcoreMesh(core_axis_name="c", subcore_axis_name="s")
    return pl.kernel(
        _sc_scores_kernel,
        mesh=mesh,
        compiler_params=pltpu.CompilerParams(
            needs_layout_passes=False, use_tc_tiling_on_sc=False),
        out_type=[
            jax.ShapeDtypeStruct((B,), jnp.float32),
            jax.ShapeDtypeStruct((NEG * B,), jnp.float32),
        ],
        scratch_types=[
            pltpu.VMEM((BPW,), jnp.int32),
            pltpu.VMEM((BPW,), jnp.int32),
            pltpu.VMEM((NEG * BPW,), jnp.int32),
            pltpu.VMEM((CHUNK, D), jnp.float32),
            pltpu.VMEM((CHUNK, D), jnp.float32),
            pltpu.VMEM((NEG * CHUNK, D), jnp.float32),
            pltpu.VMEM((CHUNK, D), jnp.float32),
            pltpu.VMEM((CHUNK, D), jnp.float32),
            pltpu.VMEM((NEG * CHUNK, D), jnp.float32),
            pltpu.VMEM((CHUNK,), jnp.float32),
            pltpu.VMEM((NEG * CHUNK,), jnp.float32),
            pltpu.SemaphoreType.DMA,
            pltpu.SemaphoreType.DMA,
        ],
    )(pos_u, pos_v, neg_vT, u_emb, v_emb)


def _loss_body(pos_ref, neg_ref, out_ref):
    p = pos_ref[...]
    n = neg_ref[...]
    ls_p = jnp.minimum(p, 0.0) - jnp.log1p(jnp.exp(-jnp.abs(p)))
    ls_n = jnp.minimum(-n, 0.0) - jnp.log1p(jnp.exp(-jnp.abs(n)))
    out_ref[0, 0] = -(jnp.sum(ls_p) + jnp.sum(ls_n))


@jax.jit
def _tc_loss(pos_s, neg_s):
    out = pl.pallas_call(
        _loss_body,
        out_shape=jax.ShapeDtypeStruct((1, 1), jnp.float32),
        out_specs=pl.BlockSpec(memory_space=pltpu.SMEM),
    )(pos_s, neg_s)
    return out[0, 0]


def kernel(pos_u, pos_v, neg_v, u_emb, v_emb):
    pos_u = pos_u.astype(jnp.int32)
    pos_v = pos_v.astype(jnp.int32)
    neg_vT = neg_v.astype(jnp.int32).T.reshape(NEG * B)
    pos_s, neg_sT = _sc_scores(pos_u, pos_v, neg_vT, u_emb, v_emb)
    return _tc_loss(pos_s.reshape(B // 128, 128),
                    neg_sT.reshape(NEG * B // 128, 128))
